# baseline (speedup 1.0000x reference)
import jax
import jax.numpy as jnp
from jax.experimental import pallas as pl

_NEIGHBOR_K = 3
_SELECT_RATIO = 0.02
_BN = 896
_LANE = 128


def _make_sim_kernel(m_logical):
    n_chunks = -(-m_logical // _LANE)
    rem = m_logical - (n_chunks - 1) * _LANE

    def _sim_kernel(u_ref, v_ref, out_ref):
        u = u_ref[:, :]
        v = v_ref[0]
        s = jnp.dot(u, v, preferred_element_type=jnp.float32)

        def chunk(i):
            c = s[:, i * _LANE:(i + 1) * _LANE]
            if i == n_chunks - 1 and rem != _LANE:
                lane = jax.lax.broadcasted_iota(jnp.int32, c.shape, 1)
                c = jnp.where(lane < rem, c, -jnp.inf)
            return c

        c0, c1, c2 = chunk(0), chunk(1), chunk(2)
        p, q = jnp.maximum(c0, c1), jnp.minimum(c0, c1)
        r, a3 = jnp.maximum(q, c2), jnp.minimum(q, c2)
        a1, a2 = jnp.maximum(p, r), jnp.minimum(p, r)
        for i in range(3, n_chunks):
            c = chunk(i)
            hi, lo = jnp.maximum(a1, c), jnp.minimum(a1, c)
            a1 = hi
            hi2, lo2 = jnp.maximum(a2, lo), jnp.minimum(a2, lo)
            a2 = hi2
            a3 = jnp.maximum(a3, lo2)
        col = jax.lax.broadcasted_iota(jnp.int32, a1.shape, 1)
        total = jnp.zeros((a1.shape[0], 1), jnp.float32)
        for _ in range(_NEIGHBOR_K):
            m = jnp.max(a1, axis=1, keepdims=True)
            total = total + m
            idx = jnp.min(jnp.where(a1 == m, col, _LANE), axis=1, keepdims=True)
            hit = col == idx
            a1 = jnp.where(hit, a2, a1)
            a2 = jnp.where(hit, a3, a2)
            a3 = jnp.where(hit, -jnp.inf, a3)
        out_ref[:, :] = total

    return _sim_kernel


def _compute_sims(un, vn, m_logical):
    k_cls, c, m_pad = vn.shape
    n = un.shape[0]
    nt = n // _BN
    out = pl.pallas_call(
        _make_sim_kernel(m_logical),
        grid=(k_cls, nt),
        in_specs=[
            pl.BlockSpec((_BN, c), lambda j, t: (t, 0)),
            pl.BlockSpec((1, c, m_pad), lambda j, t: (j, 0, 0)),
        ],
        out_specs=pl.BlockSpec((_BN, 1), lambda j, t: (j * nt + t, 0)),
        out_shape=jax.ShapeDtypeStruct((k_cls * n, 1), jnp.float32),
    )(un, vn)
    return out.reshape(k_cls, n)


def kernel(x1, x2):
    b, c, h, w = x1.shape
    n = b * h * w
    raw = jnp.transpose(x1, (1, 0, 2, 3)).reshape(c, -1)
    un = raw.T
    un = un / jnp.linalg.norm(un, ord=2, axis=1, keepdims=True)
    vn = jnp.stack(
        [
            x2[j] / jnp.linalg.norm(x2[j], ord=2, axis=0, keepdims=True)
            for j in range(x2.shape[0])
        ]
    )
    m = vn.shape[2]
    m_pad = (-(-m // _LANE)) * _LANE
    vnb = jnp.pad(vn.astype(jnp.bfloat16), ((0, 0), (0, 0), (0, m_pad - m)))
    sims = _compute_sims(un.astype(jnp.bfloat16), vnb, m)
    select_num = int(n * _SELECT_RATIO)
    _, sel = jax.lax.top_k(sims, select_num)
    g = jnp.transpose(raw.T[sel], (0, 2, 1))
    return jnp.concatenate([x2, g], axis=2)
